# baseline (speedup 1.0000x reference)
import functools

import jax
import jax.numpy as jnp
from jax import lax
from jax.experimental import pallas as pl
from jax.experimental.pallas import tpu as pltpu
from jax.experimental.pallas import tpu_sc as plsc

_NC = 2
_NS = 16
_NBUF = 2


def _sc_body(n_runs, topk, cb, sbs, kv_dim, rope_dim, ftab_cols, stab_cols,
             topk_hbm, ftab_hbm, stab_hbm, seq_hbm, kv_hbm, rope_hbm,
             out_rope, out_kv,
             topk_v, ftab_v, stab_v, seq_v, buf_kv, buf_rope,
             skv_b, srope_b, zkv_v, zrope_v, sem_i, sem_o):
    nw = _NC * _NS
    runs_per_w = n_runs // nw
    wid = lax.axis_index("s") * _NC + lax.axis_index("c")

    skv = [skv_b.at[pl.ds(i * sbs, sbs), :] for i in range(_NBUF)]
    srope = [srope_b.at[pl.ds(i * sbs, sbs), :] for i in range(_NBUF)]
    sem_in = [sem_i.at[i] for i in range(_NBUF)]
    sem_out = [sem_o.at[i] for i in range(_NBUF)]

    stage = [(topk_hbm, topk_v), (ftab_hbm, ftab_v), (stab_hbm, stab_v),
             (seq_hbm, seq_v)]
    for src_ref, dst_ref in stage:
        pltpu.async_copy(src_ref, dst_ref.at[pl.ds(0, src_ref.shape[0])],
                         sem_i.at[_NBUF])
    for src_ref, dst_ref in stage:
        pltpu.make_async_copy(src_ref,
                              dst_ref.at[pl.ds(0, src_ref.shape[0])],
                              sem_i.at[_NBUF]).wait()

    runs_per_cb = cb // sbs
    runs = []

    for k in range(runs_per_w):
        r = k * nw + wid
        b = r // topk
        t = r % topk
        idx = topk_v[pl.ds(r, 16)][0]
        src_blk = ftab_v[pl.ds(b * ftab_cols + idx // runs_per_cb, 16)][0]
        src = src_blk * cb + (idx % runs_per_cb) * sbs
        dst_blk = stab_v[pl.ds(b * stab_cols + t // runs_per_cb, 16)][0]
        dst = dst_blk * cb + (t % runs_per_cb) * sbs
        drow = (t % runs_per_cb) * sbs
        nv = jnp.clip(seq_v[pl.ds(b, 16)][0] - idx * sbs, 0, sbs)
        runs.append((src, dst, dst_blk, drow, nv))

    def issue_in(k):
        src, dst, dblk, drow, nv = runs[k]
        p = k % _NBUF

        @pl.when(nv > 0)
        def _():
            pltpu.async_copy(kv_hbm.at[pl.ds(src, sbs), :], skv[p], sem_in[p])
            pltpu.async_copy(rope_hbm.at[pl.ds(src, sbs), :], srope[p],
                             sem_in[p])

    def wait_in(k):
        src, dst, dblk, drow, nv = runs[k]
        p = k % _NBUF

        @pl.when(nv > 0)
        def _():
            pltpu.make_async_copy(kv_hbm.at[pl.ds(0, sbs), :], skv[p],
                                  sem_in[p]).wait()
            pltpu.make_async_copy(rope_hbm.at[pl.ds(0, sbs), :], srope[p],
                                  sem_in[p]).wait()

    def issue_out(k):
        src, dst, dblk, drow, nv = runs[k]
        p = k % _NBUF

        @pl.when(nv > 0)
        def _():
            pltpu.async_copy(skv[p], out_kv.at[pl.ds(dst, sbs), :],
                             sem_out[p])
            pltpu.async_copy(srope[p],
                             out_rope.at[dblk, pl.ds(drow, sbs), :],
                             sem_out[p])

        @pl.when(nv <= 0)
        def _():
            pltpu.async_copy(zkv_v, out_kv.at[pl.ds(dst, sbs), :], sem_out[p])
            pltpu.async_copy(zrope_v,
                             out_rope.at[dblk, pl.ds(drow, sbs), :],
                             sem_out[p])

    def wait_out(k):
        src, dst, dblk, drow, nv = runs[k]
        p = k % _NBUF
        pltpu.make_async_copy(zkv_v, out_kv.at[pl.ds(dst, sbs), :],
                              sem_out[p]).wait()
        pltpu.make_async_copy(zrope_v, out_rope.at[dblk, pl.ds(drow, sbs), :],
                              sem_out[p]).wait()

    for k in range(min(_NBUF, runs_per_w)):
        issue_in(k)

    zeros16 = jnp.zeros((16,), jnp.float32)

    def zbody(j, carry):
        for c in range(kv_dim // 16):
            zkv_v[j, pl.ds(c * 16, 16)] = zeros16
        for c in range(rope_dim // 16):
            zrope_v[j, pl.ds(c * 16, 16)] = zeros16
        return carry

    lax.fori_loop(0, sbs, zbody, 0)

    for k in range(runs_per_w):
        wait_in(k)
        issue_out(k)
        j = k + 1
        if j < runs_per_w and j >= _NBUF:
            wait_out(j - _NBUF)
            issue_in(j)
    for k in range(max(0, runs_per_w - _NBUF), runs_per_w):
        wait_out(k)

    for src, dst, dblk, drow, nv in runs:

        @pl.when((nv > 0) & (nv < sbs))
        def _():
            nv8 = (nv // 8) * 8
            s = nv - nv8
            zstart = nv8 + jnp.where(s > 0, 8, 0)

            zlen = sbs - zstart
            off = zstart
            c = sbs // 2
            while c >= 8:
                bit = (zlen // c) % 2
                cc = c

                @pl.when(bit == 1)
                def _():
                    pltpu.sync_copy(zkv_v.at[pl.ds(0, cc), :],
                                    out_kv.at[pl.ds(dst + off, cc), :])
                    pltpu.sync_copy(
                        zrope_v.at[pl.ds(0, cc), :],
                        out_rope.at[dblk, pl.ds(drow + off, cc), :])

                off = off + bit * c
                c //= 2

            @pl.when(s > 0)
            def _():
                pltpu.sync_copy(kv_hbm.at[pl.ds(src + nv8, 8), :], buf_kv)
                pltpu.sync_copy(rope_hbm.at[pl.ds(src + nv8, 8), :], buf_rope)

                def zrow(j, carry):
                    for c in range(kv_dim // 16):
                        buf_kv[j, pl.ds(c * 16, 16)] = zeros16
                    for c in range(rope_dim // 16):
                        buf_rope[j, pl.ds(c * 16, 16)] = zeros16
                    return carry

                lax.fori_loop(s, 8, zrow, 0)
                pltpu.sync_copy(buf_kv, out_kv.at[pl.ds(dst + nv8, 8), :])
                pltpu.sync_copy(
                    buf_rope, out_rope.at[dblk, pl.ds(drow + nv8, 8), :])


def kernel(selection_k_rope, selection_kv_cache, selection_kv_block_table,
           selection_kv_block_status, selection_topk_indices, full_k_rope,
           full_kv_cache, full_kv_block_table, full_kv_actual_seq,
           full_q_actual_seq, selection_topk_block_size):
    B, TOPK = selection_topk_indices.shape
    NFB, CB, KV_DIM = full_kv_cache.shape
    ROPE = full_k_rope.shape[-1]
    NSB = selection_kv_cache.shape[0]
    SBS = (NSB // B) * CB // TOPK
    N_RUNS = B * TOPK

    kv_flat = full_kv_cache.reshape(NFB * CB, KV_DIM)
    rope_flat = full_k_rope.reshape(NFB * CB, ROPE)
    topk_flat = selection_topk_indices.reshape(-1).astype(jnp.int32)
    ftab_flat = full_kv_block_table.reshape(-1).astype(jnp.int32)
    stab_flat = selection_kv_block_table.reshape(-1).astype(jnp.int32)
    seq = full_kv_actual_seq.reshape(-1).astype(jnp.int32)

    mesh = plsc.VectorSubcoreMesh(core_axis_name="c", subcore_axis_name="s",
                                  num_cores=_NC, num_subcores=_NS)
    body = functools.partial(_sc_body, N_RUNS, TOPK, CB, SBS, KV_DIM, ROPE,
                             full_kv_block_table.shape[1],
                             selection_kv_block_table.shape[1])
    pad16 = lambda n: (n + 16 + 7) // 8 * 8
    out_rope, out_kv = pl.kernel(
        body,
        out_type=[
            jax.ShapeDtypeStruct((NSB, CB, ROPE), jnp.float32),
            jax.ShapeDtypeStruct((NSB * CB, KV_DIM), jnp.float32),
        ],
        mesh=mesh,
        scratch_types=[
            pltpu.VMEM((pad16(topk_flat.shape[0]),), jnp.int32),
            pltpu.VMEM((pad16(ftab_flat.shape[0]),), jnp.int32),
            pltpu.VMEM((pad16(stab_flat.shape[0]),), jnp.int32),
            pltpu.VMEM((pad16(seq.shape[0]),), jnp.int32),
            pltpu.VMEM((8, KV_DIM), jnp.float32),
            pltpu.VMEM((8, ROPE), jnp.float32),
            pltpu.VMEM((_NBUF * SBS, KV_DIM), jnp.float32),
            pltpu.VMEM((_NBUF * SBS, ROPE), jnp.float32),
            pltpu.VMEM((SBS, KV_DIM), jnp.float32),
            pltpu.VMEM((SBS, ROPE), jnp.float32),
            pltpu.SemaphoreType.DMA((_NBUF + 1,)),
            pltpu.SemaphoreType.DMA((_NBUF,)),
        ],
    )(topk_flat, ftab_flat, stab_flat, seq, kv_flat, rope_flat)

    return (out_rope, out_kv.reshape(NSB, CB, KV_DIM))


# speedup vs baseline: 1.0044x; 1.0044x over previous
import functools

import jax
import jax.numpy as jnp
from jax import lax
from jax.experimental import pallas as pl
from jax.experimental.pallas import tpu as pltpu
from jax.experimental.pallas import tpu_sc as plsc

_NC = 2
_NS = 16
_NBUF = 2


def _sc_body(n_runs, topk, cb, sbs, kv_dim, rope_dim, ftab_cols, stab_cols,
             topk_hbm, ftab_hbm, stab_hbm, seq_hbm, kv_hbm, rope_hbm,
             out_rope, out_kv,
             topk_v, ftab_v, stab_v, seq_v, buf_kv, buf_rope,
             skv_b, srope_b, zkv_v, zrope_v, sem_i, sem_o):
    nw = _NC * _NS
    runs_per_w = n_runs // nw
    wid = lax.axis_index("s") * _NC + lax.axis_index("c")

    skv = [skv_b.at[pl.ds(i * sbs, sbs), :] for i in range(_NBUF)]
    srope = [srope_b.at[pl.ds(i * sbs, sbs), :] for i in range(_NBUF)]
    sem_in = [sem_i.at[i] for i in range(_NBUF)]
    sem_out = [sem_o.at[i] for i in range(_NBUF)]

    stage = [(topk_hbm, topk_v), (ftab_hbm, ftab_v), (stab_hbm, stab_v),
             (seq_hbm, seq_v)]
    for src_ref, dst_ref in stage:
        pltpu.async_copy(src_ref, dst_ref.at[pl.ds(0, src_ref.shape[0])],
                         sem_i.at[_NBUF])
    for src_ref, dst_ref in stage:
        pltpu.make_async_copy(src_ref,
                              dst_ref.at[pl.ds(0, src_ref.shape[0])],
                              sem_i.at[_NBUF]).wait()

    runs_per_cb = cb // sbs
    runs = []

    for k in range(runs_per_w):
        r = k * nw + wid
        b = r // topk
        t = r % topk
        idx = topk_v[pl.ds(r, 16)][0]
        src_blk = ftab_v[pl.ds(b * ftab_cols + idx // runs_per_cb, 16)][0]
        src = src_blk * cb + (idx % runs_per_cb) * sbs
        dst_blk = stab_v[pl.ds(b * stab_cols + t // runs_per_cb, 16)][0]
        dst = dst_blk * cb + (t % runs_per_cb) * sbs
        nv = jnp.clip(seq_v[pl.ds(b, 16)][0] - idx * sbs, 0, sbs)
        runs.append((src, dst, nv))

    def issue_in(k):
        src, dst, nv = runs[k]
        p = k % _NBUF

        @pl.when(nv > 0)
        def _():
            pltpu.async_copy(kv_hbm.at[pl.ds(src, sbs), :], skv[p], sem_in[p])
            pltpu.async_copy(rope_hbm.at[pl.ds(src, sbs), :], srope[p],
                             sem_in[p])

    def wait_in(k):
        src, dst, nv = runs[k]
        p = k % _NBUF

        @pl.when(nv > 0)
        def _():
            pltpu.make_async_copy(kv_hbm.at[pl.ds(0, sbs), :], skv[p],
                                  sem_in[p]).wait()
            pltpu.make_async_copy(rope_hbm.at[pl.ds(0, sbs), :], srope[p],
                                  sem_in[p]).wait()

    def issue_out(k):
        src, dst, nv = runs[k]
        p = k % _NBUF

        @pl.when(nv > 0)
        def _():
            pltpu.async_copy(skv[p], out_kv.at[pl.ds(dst, sbs), :],
                             sem_out[p])
            pltpu.async_copy(srope[p], out_rope.at[pl.ds(dst, sbs), :],
                             sem_out[p])

        @pl.when(nv <= 0)
        def _():
            pltpu.async_copy(zkv_v, out_kv.at[pl.ds(dst, sbs), :], sem_out[p])
            pltpu.async_copy(zrope_v, out_rope.at[pl.ds(dst, sbs), :],
                             sem_out[p])

    def wait_out(k):
        src, dst, nv = runs[k]
        p = k % _NBUF
        pltpu.make_async_copy(zkv_v, out_kv.at[pl.ds(dst, sbs), :],
                              sem_out[p]).wait()
        pltpu.make_async_copy(zrope_v, out_rope.at[pl.ds(dst, sbs), :],
                              sem_out[p]).wait()

    for k in range(min(_NBUF, runs_per_w)):
        issue_in(k)

    zeros16 = jnp.zeros((16,), jnp.float32)

    def zbody(j, carry):
        for c in range(kv_dim // 16):
            zkv_v[j, pl.ds(c * 16, 16)] = zeros16
        for c in range(rope_dim // 16):
            zrope_v[j, pl.ds(c * 16, 16)] = zeros16
        return carry

    lax.fori_loop(0, sbs, zbody, 0)

    def fixup(k):
        src, dst, nv = runs[k]

        @pl.when((nv > 0) & (nv < sbs))
        def _():
            nv8 = (nv // 8) * 8
            s = nv - nv8
            zstart = nv8 + jnp.where(s > 0, 8, 0)

            zlen = sbs - zstart
            off = zstart
            c = sbs // 2
            while c >= 8:
                bit = (zlen // c) % 2
                cc = c

                @pl.when(bit == 1)
                def _():
                    pltpu.sync_copy(zkv_v.at[pl.ds(0, cc), :],
                                    out_kv.at[pl.ds(dst + off, cc), :])
                    pltpu.sync_copy(zrope_v.at[pl.ds(0, cc), :],
                                    out_rope.at[pl.ds(dst + off, cc), :])

                off = off + bit * c
                c //= 2

            @pl.when(s > 0)
            def _():
                pltpu.sync_copy(kv_hbm.at[pl.ds(src + nv8, 8), :], buf_kv)
                pltpu.sync_copy(rope_hbm.at[pl.ds(src + nv8, 8), :], buf_rope)

                def zrow(j, carry):
                    for c in range(kv_dim // 16):
                        buf_kv[j, pl.ds(c * 16, 16)] = zeros16
                    for c in range(rope_dim // 16):
                        buf_rope[j, pl.ds(c * 16, 16)] = zeros16
                    return carry

                lax.fori_loop(s, 8, zrow, 0)
                pltpu.sync_copy(buf_kv, out_kv.at[pl.ds(dst + nv8, 8), :])
                pltpu.sync_copy(buf_rope, out_rope.at[pl.ds(dst + nv8, 8), :])

    for k in range(runs_per_w):
        wait_in(k)
        issue_out(k)
        j = k + 1
        if j < runs_per_w and j >= _NBUF:
            wait_out(j - _NBUF)
            issue_in(j)
            fixup(j - _NBUF)
    for k in range(max(0, runs_per_w - _NBUF), runs_per_w):
        wait_out(k)
        fixup(k)


def kernel(selection_k_rope, selection_kv_cache, selection_kv_block_table,
           selection_kv_block_status, selection_topk_indices, full_k_rope,
           full_kv_cache, full_kv_block_table, full_kv_actual_seq,
           full_q_actual_seq, selection_topk_block_size):
    B, TOPK = selection_topk_indices.shape
    NFB, CB, KV_DIM = full_kv_cache.shape
    ROPE = full_k_rope.shape[-1]
    NSB = selection_kv_cache.shape[0]
    SBS = (NSB // B) * CB // TOPK
    N_RUNS = B * TOPK

    kv_flat = full_kv_cache.reshape(NFB * CB, KV_DIM)
    rope_flat = full_k_rope.reshape(NFB * CB, ROPE)
    topk_flat = selection_topk_indices.reshape(-1).astype(jnp.int32)
    ftab_flat = full_kv_block_table.reshape(-1).astype(jnp.int32)
    stab_flat = selection_kv_block_table.reshape(-1).astype(jnp.int32)
    seq = full_kv_actual_seq.reshape(-1).astype(jnp.int32)

    mesh = plsc.VectorSubcoreMesh(core_axis_name="c", subcore_axis_name="s",
                                  num_cores=_NC, num_subcores=_NS)
    body = functools.partial(_sc_body, N_RUNS, TOPK, CB, SBS, KV_DIM, ROPE,
                             full_kv_block_table.shape[1],
                             selection_kv_block_table.shape[1])
    pad16 = lambda n: (n + 16 + 7) // 8 * 8
    out_rope, out_kv = pl.kernel(
        body,
        out_type=[
            jax.ShapeDtypeStruct((NSB * CB, ROPE), jnp.float32),
            jax.ShapeDtypeStruct((NSB * CB, KV_DIM), jnp.float32),
        ],
        mesh=mesh,
        scratch_types=[
            pltpu.VMEM((pad16(topk_flat.shape[0]),), jnp.int32),
            pltpu.VMEM((pad16(ftab_flat.shape[0]),), jnp.int32),
            pltpu.VMEM((pad16(stab_flat.shape[0]),), jnp.int32),
            pltpu.VMEM((pad16(seq.shape[0]),), jnp.int32),
            pltpu.VMEM((8, KV_DIM), jnp.float32),
            pltpu.VMEM((8, ROPE), jnp.float32),
            pltpu.VMEM((_NBUF * SBS, KV_DIM), jnp.float32),
            pltpu.VMEM((_NBUF * SBS, ROPE), jnp.float32),
            pltpu.VMEM((SBS, KV_DIM), jnp.float32),
            pltpu.VMEM((SBS, ROPE), jnp.float32),
            pltpu.SemaphoreType.DMA((_NBUF + 1,)),
            pltpu.SemaphoreType.DMA((_NBUF,)),
        ],
    )(topk_flat, ftab_flat, stab_flat, seq, kv_flat, rope_flat)

    return (out_rope.reshape(NSB, CB, ROPE), out_kv.reshape(NSB, CB, KV_DIM))
